# baseline (speedup 1.0000x reference)
import functools

import jax
import jax.numpy as jnp
from jax import lax
from jax.experimental import pallas as pl
from jax.experimental.pallas import tpu as pltpu
from jax.experimental.pallas import tpu_sc as plsc

_THRESH = 0.35667494393873245
_N_MIN = 131072
_IGNORE = 255

_HBLK = 64

_NSC = 2
_NWORK = 32
_WPB = 16

_LN2 = 0.6931471805599453
_LOG2_COEF = (
    1.442693655e+00, -7.212951469e-01, 4.801732820e-01, -3.555206404e-01,
    2.668084347e-01, -1.814503041e-01, 9.679993446e-02, -3.375161495e-02,
    5.542417541e-03,
)


def _ce_body(x_ref, lbl_ref):
    x = x_ref[0]
    lbl = lbl_ref[0]
    s = jnp.sum(jnp.exp(jnp.clip(x, -60.0, 60.0)), axis=0)
    cidx = lax.broadcasted_iota(jnp.int32, x.shape, 0)
    x_lbl = jnp.sum(jnp.where(cidx == lbl[None], x, 0.0), axis=0)
    loss = jnp.log(s) - x_lbl
    return jnp.where(lbl == _IGNORE, 0.0, loss)


def _stats_kernel(x_ref, lbl_ref, sum_ref, cnt_ref):
    i = pl.program_id(0)

    @pl.when(i == 0)
    def _init():
        sum_ref[...] = jnp.zeros((1, 1), jnp.float32)
        cnt_ref[...] = jnp.zeros((1, 1), jnp.float32)

    loss = _ce_body(x_ref, lbl_ref)
    gt = loss > _THRESH
    sum_ref[...] += jnp.sum(jnp.where(gt, loss, 0.0))[None, None]
    cnt_ref[...] += jnp.sum(gt.astype(jnp.float32))[None, None]


def _loss_kernel(x_ref, lbl_ref, loss_ref):
    loss_ref[0] = _ce_body(x_ref, lbl_ref)


def _topk_kernel(loss_ref, out_ref):
    loss = jnp.maximum(loss_ref[...], 0.0)
    bits = lax.bitcast_convert_type(loss, jnp.int32)
    k0 = jnp.int32(_N_MIN)

    def body(r, carry):
        i = 31 - r
        prefix, k = carry
        pat = lax.shift_right_logical(prefix, i) | 1
        hit = lax.shift_right_logical(bits, i) == pat
        cnt1 = jnp.sum(hit.astype(jnp.int32))
        take = cnt1 >= k
        prefix = jnp.where(take, prefix | (1 << i), prefix)
        k = jnp.where(take, k, k - cnt1)
        return prefix, k

    prefix, _ = lax.fori_loop(0, 32, body, (jnp.int32(0), k0))
    t = lax.bitcast_convert_type(prefix, jnp.float32)
    gt = bits > prefix
    cnt_gt = jnp.sum(gt.astype(jnp.float32))
    sum_gt = jnp.sum(jnp.where(gt, loss, 0.0))
    kf = jnp.float32(_N_MIN)
    out_ref[...] = ((sum_gt + t * (kf - cnt_gt)) / kf)[None, None]


def _sc_stats_kernel(x_hbm, lbl_hbm, out_hbm, xbuf, lblbuf, obuf):
    n, c, h, w = x_hbm.shape
    wid = lax.axis_index("s") * 2 + lax.axis_index("c")
    nb = (n - _NSC) + wid // _WPB
    sacc = jnp.zeros((16,), jnp.float32)
    cacc = jnp.zeros((16,), jnp.float32)

    def vbody(v, carry):
        sa, ca = carry
        row = v // 16
        b = pl.multiple_of((v % 16) * 16, 16)
        lblv = lblbuf[row, pl.ds(b, 16)]
        s = jnp.zeros((16,), jnp.float32)
        xl = jnp.zeros((16,), jnp.float32)
        for ci in range(c):
            xc = xbuf[ci, row, pl.ds(b, 16)]
            s = s + jnp.exp(jnp.clip(xc, -60.0, 60.0))
            xl = jnp.where(lblv == ci, xc, xl)
        bits = lax.bitcast_convert_type(s, jnp.int32)
        ex = ((bits >> 23) & 0xFF) - 127
        mb = (bits & 0x7FFFFF) | (127 << 23)
        t = lax.bitcast_convert_type(mb, jnp.float32) - 1.0
        acc = jnp.full((16,), _LOG2_COEF[-1], jnp.float32)
        for coef in _LOG2_COEF[-2::-1]:
            acc = acc * t + coef
        logs = (ex.astype(jnp.float32) + t * acc) * _LN2
        loss = logs - xl
        loss = jnp.where(lblv == _IGNORE, 0.0, loss)
        gt = loss > _THRESH
        sa = sa + jnp.where(gt, loss, 0.0)
        ca = ca + jnp.where(gt, 1.0, 0.0)
        return sa, ca

    nchunks = (h // 8) * (w // 256) // _WPB
    for j in range(nchunks):
        r = (wid % _WPB) * nchunks + j
        h0 = (r // 2) * 8
        w0 = (r % 2) * 256
        pltpu.sync_copy(
            x_hbm.at[nb, :, pl.ds(h0, 8), pl.ds(w0, 256)], xbuf)
        pltpu.sync_copy(lbl_hbm.at[nb, pl.ds(h0, 8), pl.ds(w0, 256)], lblbuf)
        sacc, cacc = lax.fori_loop(0, 8 * 256 // 16, vbody, (sacc, cacc))

    obuf[0] = sacc
    obuf[1] = cacc
    pltpu.sync_copy(obuf, out_hbm.at[pl.ds(wid * 2, 2)])


def kernel(logits, labels):
    n, c, h, w = logits.shape
    lbl = labels.astype(jnp.int32)
    nsteps = h // _HBLK
    grid = (n * nsteps,)

    def xmap(i):
        return (i // nsteps, 0, i % nsteps, 0)

    def lmap(i):
        return (i // nsteps, i % nsteps, 0)

    sc_out = pl.kernel(
        _sc_stats_kernel,
        mesh=plsc.VectorSubcoreMesh(core_axis_name="c", subcore_axis_name="s"),
        out_type=jax.ShapeDtypeStruct((2 * _NWORK, 16), jnp.float32),
        scratch_types=[
            pltpu.VMEM((c, 8, 256), jnp.float32),
            pltpu.VMEM((8, 256), jnp.int32),
            pltpu.VMEM((2, 16), jnp.float32),
        ],
        compiler_params=pltpu.CompilerParams(use_tc_tiling_on_sc=True),
    )(logits, lbl)

    n_tc = n - _NSC
    sum_gt, cnt_gt = pl.pallas_call(
        _stats_kernel,
        grid=(n_tc * nsteps,),
        in_specs=[
            pl.BlockSpec((1, c, _HBLK, w), xmap),
            pl.BlockSpec((1, _HBLK, w), lmap),
        ],
        out_specs=[
            pl.BlockSpec((1, 1), lambda i: (0, 0)),
            pl.BlockSpec((1, 1), lambda i: (0, 0)),
        ],
        out_shape=[
            jax.ShapeDtypeStruct((1, 1), jnp.float32),
            jax.ShapeDtypeStruct((1, 1), jnp.float32),
        ],
    )(logits, lbl)

    sc_pairs = sc_out.reshape(_NWORK, 2, 16)
    s = sum_gt[0, 0] + jnp.sum(sc_pairs[:, 0, :])
    cnt = cnt_gt[0, 0] + jnp.sum(sc_pairs[:, 1, :])
    cond = cnt > _N_MIN + 0.5
    mean_thresh = s / jnp.maximum(cnt, 1.0)

    def fallback(_):
        loss = pl.pallas_call(
            _loss_kernel,
            grid=grid,
            in_specs=[
                pl.BlockSpec((1, c, _HBLK, w), xmap),
                pl.BlockSpec((1, _HBLK, w), lmap),
            ],
            out_specs=pl.BlockSpec((1, _HBLK, w), lmap),
            out_shape=jax.ShapeDtypeStruct((n, h, w), jnp.float32),
        )(logits, lbl)
        loss2 = loss.reshape(n * h, w)
        res = pl.pallas_call(
            _topk_kernel,
            out_shape=jax.ShapeDtypeStruct((1, 1), jnp.float32),
        )(loss2)
        return res[0, 0]

    return lax.cond(cond, lambda _: mean_thresh, fallback, None)


# speedup vs baseline: 1.1046x; 1.1046x over previous
import functools

import jax
import jax.numpy as jnp
from jax import lax
from jax.experimental import pallas as pl
from jax.experimental.pallas import tpu as pltpu

_THRESH = 0.35667494393873245
_N_MIN = 131072
_IGNORE = 255

_HBLK = 64


def _ce_body(x_ref, lbl_ref):
    x = x_ref[0]
    lbl = lbl_ref[0]
    s = jnp.sum(jnp.exp(jnp.minimum(x, 60.0)), axis=0)
    cidx = lax.broadcasted_iota(jnp.int32, x.shape, 0)
    x_lbl = jnp.sum(jnp.where(cidx == lbl[None], x, 0.0), axis=0)
    loss = jnp.log(s) - x_lbl
    return jnp.where(lbl == _IGNORE, 0.0, loss)


def _stats_kernel(x_ref, lbl_ref, sum_ref, cnt_ref):
    i = pl.program_id(0)

    @pl.when(i == 0)
    def _init():
        sum_ref[...] = jnp.zeros((1, 1), jnp.float32)
        cnt_ref[...] = jnp.zeros((1, 1), jnp.float32)

    loss = _ce_body(x_ref, lbl_ref)
    gt = loss > _THRESH
    sum_ref[...] += jnp.sum(jnp.where(gt, loss, 0.0))[None, None]
    cnt_ref[...] += jnp.sum(gt.astype(jnp.float32))[None, None]


def _loss_kernel(x_ref, lbl_ref, loss_ref):
    loss_ref[0] = _ce_body(x_ref, lbl_ref)


def _topk_kernel(loss_ref, out_ref):
    loss = jnp.maximum(loss_ref[...], 0.0)
    bits = lax.bitcast_convert_type(loss, jnp.int32)
    k0 = jnp.int32(_N_MIN)

    def body(r, carry):
        i = 31 - r
        prefix, k = carry
        pat = lax.shift_right_logical(prefix, i) | 1
        hit = lax.shift_right_logical(bits, i) == pat
        cnt1 = jnp.sum(hit.astype(jnp.int32))
        take = cnt1 >= k
        prefix = jnp.where(take, prefix | (1 << i), prefix)
        k = jnp.where(take, k, k - cnt1)
        return prefix, k

    prefix, _ = lax.fori_loop(0, 32, body, (jnp.int32(0), k0))
    t = lax.bitcast_convert_type(prefix, jnp.float32)
    gt = bits > prefix
    cnt_gt = jnp.sum(gt.astype(jnp.float32))
    sum_gt = jnp.sum(jnp.where(gt, loss, 0.0))
    kf = jnp.float32(_N_MIN)
    out_ref[...] = ((sum_gt + t * (kf - cnt_gt)) / kf)[None, None]


def kernel(logits, labels):
    n, c, h, w = logits.shape
    lbl = labels.astype(jnp.int32)
    nsteps = h // _HBLK
    grid = (n * nsteps,)

    def xmap(i):
        return (i // nsteps, 0, i % nsteps, 0)

    def lmap(i):
        return (i // nsteps, i % nsteps, 0)

    sum_gt, cnt_gt = pl.pallas_call(
        _stats_kernel,
        grid=grid,
        in_specs=[
            pl.BlockSpec((1, c, _HBLK, w), xmap),
            pl.BlockSpec((1, _HBLK, w), lmap),
        ],
        out_specs=[
            pl.BlockSpec((1, 1), lambda i: (0, 0)),
            pl.BlockSpec((1, 1), lambda i: (0, 0)),
        ],
        out_shape=[
            jax.ShapeDtypeStruct((1, 1), jnp.float32),
            jax.ShapeDtypeStruct((1, 1), jnp.float32),
        ],
    )(logits, lbl)

    s = sum_gt[0, 0]
    cnt = cnt_gt[0, 0]
    cond = cnt > _N_MIN + 0.5
    mean_thresh = s / jnp.maximum(cnt, 1.0)

    def fallback(_):
        loss = pl.pallas_call(
            _loss_kernel,
            grid=grid,
            in_specs=[
                pl.BlockSpec((1, c, _HBLK, w), xmap),
                pl.BlockSpec((1, _HBLK, w), lmap),
            ],
            out_specs=pl.BlockSpec((1, _HBLK, w), lmap),
            out_shape=jax.ShapeDtypeStruct((n, h, w), jnp.float32),
        )(logits, lbl)
        loss2 = loss.reshape(n * h, w)
        res = pl.pallas_call(
            _topk_kernel,
            out_shape=jax.ShapeDtypeStruct((1, 1), jnp.float32),
        )(loss2)
        return res[0, 0]

    return lax.cond(cond, lambda _: mean_thresh, fallback, None)


# speedup vs baseline: 1.3598x; 1.2310x over previous
import functools

import jax
import jax.numpy as jnp
from jax import lax
from jax.experimental import pallas as pl
from jax.experimental.pallas import tpu as pltpu

_THRESH = 0.35667494393873245
_N_MIN = 131072
_IGNORE = 255

_HBLK = 128


def _ce_body(x_ref, lbl_ref):
    x = x_ref[0]
    lbl = lbl_ref[0]
    s = jnp.sum(jnp.exp(jnp.minimum(x, 60.0)), axis=0)
    cidx = lax.broadcasted_iota(jnp.int32, x.shape, 0)
    x_lbl = jnp.sum(jnp.where(cidx == lbl[None], x, 0.0), axis=0)
    loss = jnp.log(s) - x_lbl
    return jnp.where(lbl == _IGNORE, 0.0, loss)


def _stats_kernel(x_ref, lbl_ref, sum_ref, cnt_ref):
    i = pl.program_id(0)

    @pl.when(i == 0)
    def _init():
        sum_ref[...] = jnp.zeros((1, 1), jnp.float32)
        cnt_ref[...] = jnp.zeros((1, 1), jnp.float32)

    loss = _ce_body(x_ref, lbl_ref)
    gt = loss > _THRESH
    sum_ref[...] += jnp.sum(jnp.where(gt, loss, 0.0))[None, None]
    cnt_ref[...] += jnp.sum(gt.astype(jnp.float32))[None, None]


def _loss_kernel(x_ref, lbl_ref, loss_ref):
    loss_ref[0] = _ce_body(x_ref, lbl_ref)


def _topk_kernel(loss_ref, out_ref):
    loss = jnp.maximum(loss_ref[...], 0.0)
    bits = lax.bitcast_convert_type(loss, jnp.int32)
    k0 = jnp.int32(_N_MIN)

    def body(r, carry):
        i = 31 - r
        prefix, k = carry
        pat = lax.shift_right_logical(prefix, i) | 1
        hit = lax.shift_right_logical(bits, i) == pat
        cnt1 = jnp.sum(hit.astype(jnp.int32))
        take = cnt1 >= k
        prefix = jnp.where(take, prefix | (1 << i), prefix)
        k = jnp.where(take, k, k - cnt1)
        return prefix, k

    prefix, _ = lax.fori_loop(0, 32, body, (jnp.int32(0), k0))
    t = lax.bitcast_convert_type(prefix, jnp.float32)
    gt = bits > prefix
    cnt_gt = jnp.sum(gt.astype(jnp.float32))
    sum_gt = jnp.sum(jnp.where(gt, loss, 0.0))
    kf = jnp.float32(_N_MIN)
    out_ref[...] = ((sum_gt + t * (kf - cnt_gt)) / kf)[None, None]


def kernel(logits, labels):
    n, c, h, w = logits.shape
    lbl = labels.astype(jnp.int32)
    nsteps = h // _HBLK
    grid = (n * nsteps,)

    def xmap(i):
        return (i // nsteps, 0, i % nsteps, 0)

    def lmap(i):
        return (i // nsteps, i % nsteps, 0)

    sum_gt, cnt_gt = pl.pallas_call(
        _stats_kernel,
        grid=grid,
        in_specs=[
            pl.BlockSpec((1, c, _HBLK, w), xmap),
            pl.BlockSpec((1, _HBLK, w), lmap),
        ],
        out_specs=[
            pl.BlockSpec((1, 1), lambda i: (0, 0)),
            pl.BlockSpec((1, 1), lambda i: (0, 0)),
        ],
        out_shape=[
            jax.ShapeDtypeStruct((1, 1), jnp.float32),
            jax.ShapeDtypeStruct((1, 1), jnp.float32),
        ],
    )(logits, lbl)

    s = sum_gt[0, 0]
    cnt = cnt_gt[0, 0]
    cond = cnt > _N_MIN + 0.5
    mean_thresh = s / jnp.maximum(cnt, 1.0)

    def fallback(_):
        loss = pl.pallas_call(
            _loss_kernel,
            grid=grid,
            in_specs=[
                pl.BlockSpec((1, c, _HBLK, w), xmap),
                pl.BlockSpec((1, _HBLK, w), lmap),
            ],
            out_specs=pl.BlockSpec((1, _HBLK, w), lmap),
            out_shape=jax.ShapeDtypeStruct((n, h, w), jnp.float32),
        )(logits, lbl)
        loss2 = loss.reshape(n * h, w)
        res = pl.pallas_call(
            _topk_kernel,
            out_shape=jax.ShapeDtypeStruct((1, 1), jnp.float32),
        )(loss2)
        return res[0, 0]

    return lax.cond(cond, lambda _: mean_thresh, fallback, None)


# speedup vs baseline: 1.4728x; 1.0831x over previous
import functools

import jax
import jax.numpy as jnp
from jax import lax
from jax.experimental import pallas as pl
from jax.experimental.pallas import tpu as pltpu

_THRESH = 0.35667494393873245
_N_MIN = 131072
_IGNORE = 255

_HBLK = 256


def _ce_body(x_ref, lbl_ref):
    x = x_ref[0]
    lbl = lbl_ref[0]
    s = jnp.sum(jnp.exp(jnp.minimum(x, 60.0)), axis=0)
    cidx = lax.broadcasted_iota(jnp.int32, x.shape, 0)
    x_lbl = jnp.sum(jnp.where(cidx == lbl[None], x, 0.0), axis=0)
    loss = jnp.log(s) - x_lbl
    return jnp.where(lbl == _IGNORE, 0.0, loss)


def _stats_kernel(x_ref, lbl_ref, sum_ref, cnt_ref):
    i = pl.program_id(0)

    @pl.when(i == 0)
    def _init():
        sum_ref[...] = jnp.zeros((1, 1), jnp.float32)
        cnt_ref[...] = jnp.zeros((1, 1), jnp.float32)

    loss = _ce_body(x_ref, lbl_ref)
    gt = loss > _THRESH
    sum_ref[...] += jnp.sum(jnp.where(gt, loss, 0.0))[None, None]
    cnt_ref[...] += jnp.sum(gt.astype(jnp.float32))[None, None]


def _loss_kernel(x_ref, lbl_ref, loss_ref):
    loss_ref[0] = _ce_body(x_ref, lbl_ref)


def _topk_kernel(loss_ref, out_ref):
    loss = jnp.maximum(loss_ref[...], 0.0)
    bits = lax.bitcast_convert_type(loss, jnp.int32)
    k0 = jnp.int32(_N_MIN)

    def body(r, carry):
        i = 31 - r
        prefix, k = carry
        pat = lax.shift_right_logical(prefix, i) | 1
        hit = lax.shift_right_logical(bits, i) == pat
        cnt1 = jnp.sum(hit.astype(jnp.int32))
        take = cnt1 >= k
        prefix = jnp.where(take, prefix | (1 << i), prefix)
        k = jnp.where(take, k, k - cnt1)
        return prefix, k

    prefix, _ = lax.fori_loop(0, 32, body, (jnp.int32(0), k0))
    t = lax.bitcast_convert_type(prefix, jnp.float32)
    gt = bits > prefix
    cnt_gt = jnp.sum(gt.astype(jnp.float32))
    sum_gt = jnp.sum(jnp.where(gt, loss, 0.0))
    kf = jnp.float32(_N_MIN)
    out_ref[...] = ((sum_gt + t * (kf - cnt_gt)) / kf)[None, None]


def kernel(logits, labels):
    n, c, h, w = logits.shape
    lbl = labels.astype(jnp.int32)
    nsteps = h // _HBLK
    grid = (n * nsteps,)

    def xmap(i):
        return (i // nsteps, 0, i % nsteps, 0)

    def lmap(i):
        return (i // nsteps, i % nsteps, 0)

    sum_gt, cnt_gt = pl.pallas_call(
        _stats_kernel,
        grid=grid,
        in_specs=[
            pl.BlockSpec((1, c, _HBLK, w), xmap),
            pl.BlockSpec((1, _HBLK, w), lmap),
        ],
        out_specs=[
            pl.BlockSpec((1, 1), lambda i: (0, 0)),
            pl.BlockSpec((1, 1), lambda i: (0, 0)),
        ],
        out_shape=[
            jax.ShapeDtypeStruct((1, 1), jnp.float32),
            jax.ShapeDtypeStruct((1, 1), jnp.float32),
        ],
    )(logits, lbl)

    s = sum_gt[0, 0]
    cnt = cnt_gt[0, 0]
    cond = cnt > _N_MIN + 0.5
    mean_thresh = s / jnp.maximum(cnt, 1.0)

    def fallback(_):
        loss = pl.pallas_call(
            _loss_kernel,
            grid=grid,
            in_specs=[
                pl.BlockSpec((1, c, _HBLK, w), xmap),
                pl.BlockSpec((1, _HBLK, w), lmap),
            ],
            out_specs=pl.BlockSpec((1, _HBLK, w), lmap),
            out_shape=jax.ShapeDtypeStruct((n, h, w), jnp.float32),
        )(logits, lbl)
        loss2 = loss.reshape(n * h, w)
        res = pl.pallas_call(
            _topk_kernel,
            out_shape=jax.ShapeDtypeStruct((1, 1), jnp.float32),
        )(loss2)
        return res[0, 0]

    return lax.cond(cond, lambda _: mean_thresh, fallback, None)
